# baseline (speedup 1.0000x reference)
import functools

import jax
import jax.numpy as jnp
from jax import lax
from jax.experimental import pallas as pl
from jax.experimental.pallas import tpu as pltpu
from jax.experimental.pallas import tpu_sc as plsc

N = 10000
E = 160000
FEAT = 256
HID = 512
NB_GRAPHS = 8

NC = 2
NS = 16
CW = 128
NCHUNK = HID // CW
NPAD = 10240
RPT = NPAD // NS
ET = E // NS
BR = 120
NBATCH = 3 * (-(-ET // (3 * BR)))
EW = E // (NC * NS)
NBD = -(-EW // CW)

_mesh = plsc.VectorSubcoreMesh(
    core_axis_name="c", subcore_axis_name="s", num_cores=NC, num_subcores=NS)


@functools.partial(
    pl.kernel,
    out_type=jax.ShapeDtypeStruct((NC, NPAD), jnp.float32),
    mesh=_mesh,
    scratch_types=[
        pltpu.VMEM_SHARED((NPAD,), jnp.float32),
        pltpu.VMEM((NBD, CW), jnp.int32),
        pltpu.VMEM((CW,), jnp.float32),
        pltpu.VMEM((RPT,), jnp.float32),
    ],
)
def _sc_degree(dst_hbm, ones_hbm, hist_hbm, hist_sh, dstv, onesv, zerov):
    cid = lax.axis_index("c")
    sid = lax.axis_index("s")
    pltpu.sync_copy(ones_hbm.at[pl.ds(0, CW)], onesv)
    pltpu.sync_copy(ones_hbm.at[pl.ds(CW, RPT)], zerov)
    pltpu.sync_copy(dst_hbm.at[cid, sid], dstv)
    pltpu.sync_copy(zerov, hist_sh.at[pl.ds(sid * RPT, RPT)])
    plsc.subcore_barrier()

    def body(i, carry):
        pltpu.sync_copy(onesv, hist_sh.at[dstv.at[i]], add=True)
        return carry

    lax.fori_loop(0, NBD, body, 0)
    plsc.subcore_barrier()
    pltpu.sync_copy(hist_sh.at[pl.ds(sid * RPT, RPT)],
                    hist_hbm.at[cid, pl.ds(sid * RPT, RPT)])


@functools.partial(
    pl.kernel,
    out_type=[jax.ShapeDtypeStruct((NPAD, CW), jnp.float32)
              for _ in range(NCHUNK)],
    mesh=_mesh,
    scratch_types=[
        pltpu.VMEM_SHARED((NPAD, CW), jnp.float32),
        [pltpu.VMEM((2, BR), jnp.int32) for _ in range(3)],
        [pltpu.VMEM((BR, CW), jnp.float32) for _ in range(3)],
        [pltpu.SemaphoreType.DMA for _ in range(3)],
        [pltpu.SemaphoreType.DMA for _ in range(3)],
    ],
)
def _sc_aggregate(h0, h1, h2, h3, sd_hbm, zeros_hbm,
                  a0, a1, a2, a3, acc_sh, ibs, gbs, sg, ss):
    cid = lax.axis_index("c")
    sid = lax.axis_index("s")
    tables = (h0, h1, h2, h3)
    outs = (a0, a1, a2, a3)

    def gather(table, slot, bi):
        pltpu.sync_copy(sd_hbm.at[sid, bi], ibs[slot])
        pltpu.async_copy(table.at[ibs[slot].at[0]], gbs[slot], sg[slot])

    for chunk in range(NCHUNK):
        @pl.when(cid == chunk // 2)
        def _run(table=tables[chunk], agg=outs[chunk]):
            row0 = sid * RPT
            for k in range(RPT // CW):
                pltpu.sync_copy(zeros_hbm, acc_sh.at[pl.ds(row0 + k * CW, CW)])
            plsc.subcore_barrier()

            for b in range(2):
                gather(table, b, b)

            def outer(bo, carry):
                for b3 in range(3):
                    bi = 3 * bo + b3
                    pltpu.make_async_copy(
                        table.at[ibs[b3].at[0]], gbs[b3], sg[b3]).wait()
                    pltpu.async_copy(
                        gbs[b3], acc_sh.at[ibs[b3].at[1]], ss[b3], add=True)
                    ns = (b3 + 2) % 3

                    @pl.when(bi + 2 < NBATCH)
                    def _():
                        def relaunch():
                            pltpu.make_async_copy(
                                gbs[ns], acc_sh.at[ibs[ns].at[1]],
                                ss[ns]).wait()
                            gather(table, ns, bi + 2)
                        if b3 == 0:
                            @pl.when(bo >= 1)
                            def _():
                                relaunch()

                            @pl.when(bo == 0)
                            def _():
                                gather(table, ns, bi + 2)
                        else:
                            relaunch()
                return carry

            lax.fori_loop(0, NBATCH // 3, outer, 0)
            for k in range(3):
                pltpu.make_async_copy(
                    gbs[k], acc_sh.at[ibs[k].at[1]], ss[k]).wait()
            plsc.subcore_barrier()
            for k in range(RPT // CW):
                sl = pl.ds(row0 + k * CW, CW)
                pltpu.sync_copy(acc_sh.at[sl], agg.at[sl])
            plsc.subcore_barrier()


RB = 1000


def _dinv_body(hist_ref, o_ref):
    o_ref[...] = lax.rsqrt(hist_ref[0] + hist_ref[1] + 1.0)


def _mm_scale_body(x_ref, w_ref, dinv_ref, o0, o1, o2, o3):
    hs = jnp.dot(x_ref[...], w_ref[...],
                 preferred_element_type=jnp.float32) * dinv_ref[...]
    for k, o in enumerate((o0, o1, o2, o3)):
        o[...] = hs[:, k * CW:(k + 1) * CW]


def _post_act(a_refs, h_refs, dinv_ref, b_ref, g_ref, be_ref):
    agg = jnp.concatenate([r[...] for r in a_refs], axis=1)
    hs = jnp.concatenate([r[...] for r in h_refs], axis=1)
    t = (agg + hs) * dinv_ref[...] + b_ref[...]
    mu = jnp.mean(t, axis=1, keepdims=True)
    d = t - mu
    var = jnp.mean(d * d, axis=1, keepdims=True)
    y = d * lax.rsqrt(var + 1e-5) * g_ref[...] + be_ref[...]
    return jnp.maximum(y, 0.0)


def _ln_mm_body(a0, a1, a2, a3, h0, h1, h2, h3, dinv_ref, b_ref, g_ref,
                be_ref, w_ref, o0, o1, o2, o3):
    act = _post_act((a0, a1, a2, a3), (h0, h1, h2, h3), dinv_ref, b_ref,
                    g_ref, be_ref)
    hs = jnp.dot(act, w_ref[...],
                 preferred_element_type=jnp.float32) * dinv_ref[...]
    for k, o in enumerate((o0, o1, o2, o3)):
        o[...] = hs[:, k * CW:(k + 1) * CW]


def _ln_pool_body(a0, a1, a2, a3, h0, h1, h2, h3, dinv_ref, b_ref, g_ref,
                  be_ref, batch_ref, out_ref, sum_ref, cnt_ref, max_ref):
    i = pl.program_id(0)

    @pl.when(i == 0)
    def _():
        sum_ref[...] = jnp.zeros_like(sum_ref)
        cnt_ref[...] = jnp.zeros_like(cnt_ref)
        max_ref[...] = jnp.full_like(max_ref, -jnp.inf)

    act = _post_act((a0, a1, a2, a3), (h0, h1, h2, h3), dinv_ref, b_ref,
                    g_ref, be_ref)
    bb = batch_ref[...]
    ids = lax.broadcasted_iota(jnp.int32, (RB, NB_GRAPHS), 1)
    mask = (bb == ids).astype(jnp.float32)
    sum_ref[...] += jnp.dot(mask.T, act, preferred_element_type=jnp.float32)
    cnt_ref[...] += jnp.dot(mask.T, jnp.ones((RB, HID), jnp.float32),
                            preferred_element_type=jnp.float32)
    for b in range(NB_GRAPHS):
        masked = jnp.where(bb == b, act, -jnp.inf)
        max_ref[b:b + 1, :] = jnp.maximum(
            max_ref[b:b + 1, :], jnp.max(masked, axis=0, keepdims=True))

    @pl.when(i == N // RB - 1)
    def _():
        mean = sum_ref[...] / jnp.maximum(cnt_ref[...], 1.0)
        out_ref[...] = jnp.concatenate([mean, max_ref[...]], axis=1)


def _chunk_specs():
    return [pl.BlockSpec((RB, CW), lambda i: (i, 0)) for _ in range(NCHUNK)]


def _chunk_shapes():
    return [jax.ShapeDtypeStruct((N, CW), jnp.float32) for _ in range(NCHUNK)]


_dinv_call = pl.pallas_call(
    _dinv_body,
    out_shape=jax.ShapeDtypeStruct((NPAD // CW, CW), jnp.float32),
    in_specs=[pl.BlockSpec((NC, NPAD // CW, CW), lambda: (0, 0, 0))],
    out_specs=pl.BlockSpec((NPAD // CW, CW), lambda: (0, 0)),
)

_mm_scale_call = pl.pallas_call(
    _mm_scale_body,
    grid=(N // RB,),
    out_shape=_chunk_shapes(),
    in_specs=[
        pl.BlockSpec((RB, FEAT), lambda i: (i, 0)),
        pl.BlockSpec((FEAT, HID), lambda i: (0, 0)),
        pl.BlockSpec((RB, 1), lambda i: (i, 0)),
    ],
    out_specs=_chunk_specs(),
)

_vec_spec = pl.BlockSpec((1, HID), lambda i: (0, 0))

_agg_in_specs = [pl.BlockSpec((RB, CW), lambda i: (i, 0))
                 for _ in range(NCHUNK)]

_ln_mm_call = pl.pallas_call(
    _ln_mm_body,
    grid=(N // RB,),
    out_shape=_chunk_shapes(),
    in_specs=(_agg_in_specs + _chunk_specs()
              + [pl.BlockSpec((RB, 1), lambda i: (i, 0)),
                 _vec_spec, _vec_spec, _vec_spec,
                 pl.BlockSpec((HID, HID), lambda i: (0, 0))]),
    out_specs=_chunk_specs(),
)

_ln_pool_call = pl.pallas_call(
    _ln_pool_body,
    grid=(N // RB,),
    out_shape=jax.ShapeDtypeStruct((NB_GRAPHS, 2 * HID), jnp.float32),
    in_specs=(_agg_in_specs + _chunk_specs()
              + [pl.BlockSpec((RB, 1), lambda i: (i, 0)),
                 _vec_spec, _vec_spec, _vec_spec,
                 pl.BlockSpec((RB, 1), lambda i: (i, 0))]),
    out_specs=pl.BlockSpec((NB_GRAPHS, 2 * HID), lambda i: (0, 0)),
    scratch_shapes=[
        pltpu.VMEM((NB_GRAPHS, HID), jnp.float32),
        pltpu.VMEM((NB_GRAPHS, HID), jnp.float32),
        pltpu.VMEM((NB_GRAPHS, HID), jnp.float32),
    ],
)


def kernel(x, edge_index, batch, W1, b1, g1, be1, W2, b2, g2, be2,
           W3, b3, g3, be3):
    src = edge_index[0].astype(jnp.int32)
    dst = edge_index[1].astype(jnp.int32)

    srcp = jnp.pad(src.reshape(NS, ET), ((0, 0), (0, NBATCH * BR - ET)),
                   constant_values=0).reshape(NS, NBATCH, BR)
    dstp = jnp.pad(dst.reshape(NS, ET), ((0, 0), (0, NBATCH * BR - ET)),
                   constant_values=N).reshape(NS, NBATCH, BR)
    sd = jnp.stack([srcp, dstp], axis=2)
    dstd = jnp.pad(dst.reshape(NC * NS, EW), ((0, 0), (0, NBD * CW - EW)),
                   constant_values=N).reshape(NC, NS, NBD, CW)

    ones_z = jnp.concatenate([jnp.ones((CW,), jnp.float32),
                              jnp.zeros((RPT,), jnp.float32)])
    zeros2d = jnp.zeros((CW, CW), jnp.float32)

    hist = _sc_degree(dstd, ones_z)
    dinv = _dinv_call(hist.reshape(NC, NPAD // CW, CW))
    dinv = dinv.reshape(-1)[:N].reshape(N, 1)

    b1r, g1r, be1r = b1.reshape(1, HID), g1.reshape(1, HID), be1.reshape(1, HID)
    b2r, g2r, be2r = b2.reshape(1, HID), g2.reshape(1, HID), be2.reshape(1, HID)
    b3r, g3r, be3r = b3.reshape(1, HID), g3.reshape(1, HID), be3.reshape(1, HID)
    batch2d = batch.astype(jnp.int32).reshape(N, 1)

    hs1 = _mm_scale_call(x, W1, dinv)
    ag1 = _sc_aggregate(*hs1, sd, zeros2d)
    hs2 = _ln_mm_call(*ag1, *hs1, dinv, b1r, g1r, be1r, W2)
    ag2 = _sc_aggregate(*hs2, sd, zeros2d)
    hs3 = _ln_mm_call(*ag2, *hs2, dinv, b2r, g2r, be2r, W3)
    ag3 = _sc_aggregate(*hs3, sd, zeros2d)
    out = _ln_pool_call(*ag3, *hs3, dinv, b3r, g3r, be3r, batch2d)
    return out


# speedup vs baseline: 1.0008x; 1.0008x over previous
import functools

import jax
import jax.numpy as jnp
from jax import lax
from jax.experimental import pallas as pl
from jax.experimental.pallas import tpu as pltpu
from jax.experimental.pallas import tpu_sc as plsc

N = 10000
E = 160000
FEAT = 256
HID = 512
NB_GRAPHS = 8

NC = 2
NS = 16
CW = 128
NCHUNK = HID // CW
NPAD = 10240
RPT = NPAD // NS
ET = E // NS
BR = 120
NBATCH = 3 * (-(-ET // (3 * BR)))
EW = E // (NC * NS)
NBD = -(-EW // CW)

_mesh = plsc.VectorSubcoreMesh(
    core_axis_name="c", subcore_axis_name="s", num_cores=NC, num_subcores=NS)


@functools.partial(
    pl.kernel,
    out_type=jax.ShapeDtypeStruct((NC, NPAD), jnp.float32),
    mesh=_mesh,
    scratch_types=[
        pltpu.VMEM_SHARED((NPAD,), jnp.float32),
        pltpu.VMEM((NBD, CW), jnp.int32),
        pltpu.VMEM((CW,), jnp.float32),
        pltpu.VMEM((RPT,), jnp.float32),
    ],
)
def _sc_degree(dst_hbm, ones_hbm, hist_hbm, hist_sh, dstv, onesv, zerov):
    cid = lax.axis_index("c")
    sid = lax.axis_index("s")
    pltpu.sync_copy(ones_hbm.at[pl.ds(0, CW)], onesv)
    pltpu.sync_copy(ones_hbm.at[pl.ds(CW, RPT)], zerov)
    pltpu.sync_copy(dst_hbm.at[cid, sid], dstv)
    pltpu.sync_copy(zerov, hist_sh.at[pl.ds(sid * RPT, RPT)])
    plsc.subcore_barrier()

    def body(i, carry):
        pltpu.sync_copy(onesv, hist_sh.at[dstv.at[i]], add=True)
        return carry

    lax.fori_loop(0, NBD, body, 0)
    plsc.subcore_barrier()
    pltpu.sync_copy(hist_sh.at[pl.ds(sid * RPT, RPT)],
                    hist_hbm.at[cid, pl.ds(sid * RPT, RPT)])


@functools.partial(
    pl.kernel,
    out_type=[jax.ShapeDtypeStruct((NPAD, CW), jnp.float32)
              for _ in range(NCHUNK)],
    mesh=_mesh,
    scratch_types=[
        pltpu.VMEM_SHARED((NPAD, CW), jnp.float32),
        [pltpu.VMEM((2, BR), jnp.int32) for _ in range(3)],
        [pltpu.VMEM((BR, CW), jnp.float32) for _ in range(3)],
        [pltpu.SemaphoreType.DMA for _ in range(3)],
        [pltpu.SemaphoreType.DMA for _ in range(3)],
    ],
)
def _sc_aggregate(h0, h1, h2, h3, sd_hbm, zeros_hbm,
                  a0, a1, a2, a3, acc_sh, ibs, gbs, sg, ss):
    cid = lax.axis_index("c")
    sid = lax.axis_index("s")
    tables = (h0, h1, h2, h3)
    outs = (a0, a1, a2, a3)

    def gather(table, slot, bi):
        pltpu.sync_copy(sd_hbm.at[sid, bi], ibs[slot])
        pltpu.async_copy(table.at[ibs[slot].at[0]], gbs[slot], sg[slot])

    for chunk in range(NCHUNK):
        @pl.when(cid == chunk // 2)
        def _run(table=tables[chunk], agg=outs[chunk]):
            row0 = sid * RPT
            for k in range(RPT // CW):
                pltpu.sync_copy(zeros_hbm, acc_sh.at[pl.ds(row0 + k * CW, CW)])
            plsc.subcore_barrier()

            for b in range(2):
                gather(table, b, b)

            def outer(bo, carry):
                for b3 in range(3):
                    bi = 3 * bo + b3
                    pltpu.make_async_copy(
                        table.at[ibs[b3].at[0]], gbs[b3], sg[b3]).wait()
                    pltpu.async_copy(
                        gbs[b3], acc_sh.at[ibs[b3].at[1]], ss[b3], add=True)
                    ns = (b3 + 2) % 3

                    @pl.when(bi + 2 < NBATCH)
                    def _():
                        def relaunch():
                            pltpu.make_async_copy(
                                gbs[ns], acc_sh.at[ibs[ns].at[1]],
                                ss[ns]).wait()
                            gather(table, ns, bi + 2)
                        if b3 == 0:
                            @pl.when(bo >= 1)
                            def _():
                                relaunch()

                            @pl.when(bo == 0)
                            def _():
                                gather(table, ns, bi + 2)
                        else:
                            relaunch()
                return carry

            lax.fori_loop(0, NBATCH // 3, outer, 0)
            for k in range(3):
                pltpu.make_async_copy(
                    gbs[k], acc_sh.at[ibs[k].at[1]], ss[k]).wait()
            plsc.subcore_barrier()
            for k in range(RPT // CW):
                sl = pl.ds(row0 + k * CW, CW)
                pltpu.sync_copy(acc_sh.at[sl], agg.at[sl])
            plsc.subcore_barrier()


RB = 2000


def _dinv_body(hist_ref, o_ref):
    o_ref[...] = lax.rsqrt(hist_ref[0] + hist_ref[1] + 1.0)


def _mm_scale_body(x_ref, w_ref, dinv_ref, o0, o1, o2, o3):
    hs = jnp.dot(x_ref[...], w_ref[...],
                 preferred_element_type=jnp.float32) * dinv_ref[...]
    for k, o in enumerate((o0, o1, o2, o3)):
        o[...] = hs[:, k * CW:(k + 1) * CW]


def _post_act(a_refs, h_refs, dinv_ref, b_ref, g_ref, be_ref):
    agg = jnp.concatenate([r[...] for r in a_refs], axis=1)
    hs = jnp.concatenate([r[...] for r in h_refs], axis=1)
    t = (agg + hs) * dinv_ref[...] + b_ref[...]
    mu = jnp.mean(t, axis=1, keepdims=True)
    d = t - mu
    var = jnp.mean(d * d, axis=1, keepdims=True)
    y = d * lax.rsqrt(var + 1e-5) * g_ref[...] + be_ref[...]
    return jnp.maximum(y, 0.0)


def _ln_mm_body(a0, a1, a2, a3, h0, h1, h2, h3, dinv_ref, b_ref, g_ref,
                be_ref, w_ref, o0, o1, o2, o3):
    act = _post_act((a0, a1, a2, a3), (h0, h1, h2, h3), dinv_ref, b_ref,
                    g_ref, be_ref)
    hs = jnp.dot(act, w_ref[...],
                 preferred_element_type=jnp.float32) * dinv_ref[...]
    for k, o in enumerate((o0, o1, o2, o3)):
        o[...] = hs[:, k * CW:(k + 1) * CW]


def _ln_pool_body(a0, a1, a2, a3, h0, h1, h2, h3, dinv_ref, b_ref, g_ref,
                  be_ref, batch_ref, out_ref, sum_ref, cnt_ref, max_ref):
    i = pl.program_id(0)

    @pl.when(i == 0)
    def _():
        sum_ref[...] = jnp.zeros_like(sum_ref)
        cnt_ref[...] = jnp.zeros_like(cnt_ref)
        max_ref[...] = jnp.full_like(max_ref, -jnp.inf)

    act = _post_act((a0, a1, a2, a3), (h0, h1, h2, h3), dinv_ref, b_ref,
                    g_ref, be_ref)
    bb = batch_ref[...]
    ids = lax.broadcasted_iota(jnp.int32, (RB, NB_GRAPHS), 1)
    mask = (bb == ids).astype(jnp.float32)
    sum_ref[...] += jnp.dot(mask.T, act, preferred_element_type=jnp.float32)
    cnt_ref[...] += jnp.dot(mask.T, jnp.ones((RB, HID), jnp.float32),
                            preferred_element_type=jnp.float32)
    for b in range(NB_GRAPHS):
        masked = jnp.where(bb == b, act, -jnp.inf)
        max_ref[b:b + 1, :] = jnp.maximum(
            max_ref[b:b + 1, :], jnp.max(masked, axis=0, keepdims=True))

    @pl.when(i == N // RB - 1)
    def _():
        mean = sum_ref[...] / jnp.maximum(cnt_ref[...], 1.0)
        out_ref[...] = jnp.concatenate([mean, max_ref[...]], axis=1)


def _chunk_specs():
    return [pl.BlockSpec((RB, CW), lambda i: (i, 0)) for _ in range(NCHUNK)]


def _chunk_shapes():
    return [jax.ShapeDtypeStruct((N, CW), jnp.float32) for _ in range(NCHUNK)]


_dinv_call = pl.pallas_call(
    _dinv_body,
    out_shape=jax.ShapeDtypeStruct((NPAD // CW, CW), jnp.float32),
    in_specs=[pl.BlockSpec((NC, NPAD // CW, CW), lambda: (0, 0, 0))],
    out_specs=pl.BlockSpec((NPAD // CW, CW), lambda: (0, 0)),
)

_mm_scale_call = pl.pallas_call(
    _mm_scale_body,
    grid=(N // RB,),
    out_shape=_chunk_shapes(),
    in_specs=[
        pl.BlockSpec((RB, FEAT), lambda i: (i, 0)),
        pl.BlockSpec((FEAT, HID), lambda i: (0, 0)),
        pl.BlockSpec((RB, 1), lambda i: (i, 0)),
    ],
    out_specs=_chunk_specs(),
)

_vec_spec = pl.BlockSpec((1, HID), lambda i: (0, 0))

_agg_in_specs = [pl.BlockSpec((RB, CW), lambda i: (i, 0))
                 for _ in range(NCHUNK)]

_ln_mm_call = pl.pallas_call(
    _ln_mm_body,
    grid=(N // RB,),
    out_shape=_chunk_shapes(),
    in_specs=(_agg_in_specs + _chunk_specs()
              + [pl.BlockSpec((RB, 1), lambda i: (i, 0)),
                 _vec_spec, _vec_spec, _vec_spec,
                 pl.BlockSpec((HID, HID), lambda i: (0, 0))]),
    out_specs=_chunk_specs(),
)

_ln_pool_call = pl.pallas_call(
    _ln_pool_body,
    grid=(N // RB,),
    out_shape=jax.ShapeDtypeStruct((NB_GRAPHS, 2 * HID), jnp.float32),
    in_specs=(_agg_in_specs + _chunk_specs()
              + [pl.BlockSpec((RB, 1), lambda i: (i, 0)),
                 _vec_spec, _vec_spec, _vec_spec,
                 pl.BlockSpec((RB, 1), lambda i: (i, 0))]),
    out_specs=pl.BlockSpec((NB_GRAPHS, 2 * HID), lambda i: (0, 0)),
    scratch_shapes=[
        pltpu.VMEM((NB_GRAPHS, HID), jnp.float32),
        pltpu.VMEM((NB_GRAPHS, HID), jnp.float32),
        pltpu.VMEM((NB_GRAPHS, HID), jnp.float32),
    ],
)


def kernel(x, edge_index, batch, W1, b1, g1, be1, W2, b2, g2, be2,
           W3, b3, g3, be3):
    src = edge_index[0].astype(jnp.int32)
    dst = edge_index[1].astype(jnp.int32)

    srcp = jnp.pad(src.reshape(NS, ET), ((0, 0), (0, NBATCH * BR - ET)),
                   constant_values=0).reshape(NS, NBATCH, BR)
    dstp = jnp.pad(dst.reshape(NS, ET), ((0, 0), (0, NBATCH * BR - ET)),
                   constant_values=N).reshape(NS, NBATCH, BR)
    sd = jnp.stack([srcp, dstp], axis=2)
    dstd = jnp.pad(dst.reshape(NC * NS, EW), ((0, 0), (0, NBD * CW - EW)),
                   constant_values=N).reshape(NC, NS, NBD, CW)

    ones_z = jnp.concatenate([jnp.ones((CW,), jnp.float32),
                              jnp.zeros((RPT,), jnp.float32)])
    zeros2d = jnp.zeros((CW, CW), jnp.float32)

    hist = _sc_degree(dstd, ones_z)
    dinv = _dinv_call(hist.reshape(NC, NPAD // CW, CW))
    dinv = dinv.reshape(-1)[:N].reshape(N, 1)

    b1r, g1r, be1r = b1.reshape(1, HID), g1.reshape(1, HID), be1.reshape(1, HID)
    b2r, g2r, be2r = b2.reshape(1, HID), g2.reshape(1, HID), be2.reshape(1, HID)
    b3r, g3r, be3r = b3.reshape(1, HID), g3.reshape(1, HID), be3.reshape(1, HID)
    batch2d = batch.astype(jnp.int32).reshape(N, 1)

    hs1 = _mm_scale_call(x, W1, dinv)
    ag1 = _sc_aggregate(*hs1, sd, zeros2d)
    hs2 = _ln_mm_call(*ag1, *hs1, dinv, b1r, g1r, be1r, W2)
    ag2 = _sc_aggregate(*hs2, sd, zeros2d)
    hs3 = _ln_mm_call(*ag2, *hs2, dinv, b2r, g2r, be2r, W3)
    ag3 = _sc_aggregate(*hs3, sd, zeros2d)
    out = _ln_pool_call(*ag3, *hs3, dinv, b3r, g3r, be3r, batch2d)
    return out
